# baseline (speedup 1.0000x reference)
import functools

import jax
import jax.numpy as jnp
from jax import lax
from jax.experimental import pallas as pl
from jax.experimental.pallas import tpu as pltpu
from jax.experimental.pallas import tpu_sc as plsc

NUM_IN = 1024
NUM_OUT = 1024
NUM_MID = 4096
FAN_IN = 16
BATCH = 1024

NC = 2
NS = 16
NW = NC * NS
RPW = NUM_MID // NW
L = 16
NCHUNK = BATCH // L


_HALF = BATCH // 2


def _pack_words(a, b):
    ba = lax.bitcast_convert_type(a, jnp.int32)
    bb = lax.bitcast_convert_type(b, jnp.int32)
    half = jnp.int32(0x8000)
    return lax.shift_right_logical(ba + half, 16) | (
        (bb + half) & jnp.int32(-65536))


def _mm1_body(w_ref, x_ref, b_ref, o_ref):
    acc = lax.dot_general(w_ref[...].astype(jnp.bfloat16),
                          x_ref[...].astype(jnp.bfloat16),
                          (((1,), (1,)), ((), ())),
                          preferred_element_type=jnp.float32)
    r = jnp.maximum(acc + b_ref[...], 0.0)
    o_ref[...] = _pack_words(r[:, :_HALF], r[:, _HALF:])


def _mm1(W_in, x, b_in):
    BM = 1024
    return pl.pallas_call(
        _mm1_body,
        grid=(NUM_MID // BM,),
        in_specs=[
            pl.BlockSpec((BM, NUM_IN), lambda i: (i, 0)),
            pl.BlockSpec((BATCH, NUM_IN), lambda i: (0, 0)),
            pl.BlockSpec((BM, 1), lambda i: (i, 0)),
        ],
        out_specs=pl.BlockSpec((BM, _HALF), lambda i: (i, 0)),
        out_shape=jax.ShapeDtypeStruct((NUM_MID, _HALF), jnp.int32),
    )(W_in, x, b_in.reshape(NUM_MID, 1))


def _mm2_body(h_ref, w_ref, b_ref, o_ref):
    w = h_ref[...]
    e = lax.bitcast_convert_type(lax.shift_left(w, 16), jnp.float32)
    o = lax.bitcast_convert_type(w & jnp.int32(-65536), jnp.float32)
    wb = w_ref[...].astype(jnp.bfloat16)
    acc_e = lax.dot_general(e.astype(jnp.bfloat16), wb,
                            (((0,), (1,)), ((), ())),
                            preferred_element_type=jnp.float32)
    acc_o = lax.dot_general(o.astype(jnp.bfloat16), wb,
                            (((0,), (1,)), ((), ())),
                            preferred_element_type=jnp.float32)
    o_ref[:_HALF, :] = acc_e + b_ref[...]
    o_ref[_HALF:, :] = acc_o + b_ref[...]


def _mm2(h2w, W_out, b_out):
    BO = 256
    return pl.pallas_call(
        _mm2_body,
        grid=(NUM_OUT // BO,),
        in_specs=[
            pl.BlockSpec((NUM_MID, _HALF), lambda i: (0, 0)),
            pl.BlockSpec((BO, NUM_MID), lambda i: (i, 0)),
            pl.BlockSpec((1, BO), lambda i: (0, i)),
        ],
        out_specs=pl.BlockSpec((BATCH, BO), lambda i: (0, i)),
        out_shape=jax.ShapeDtypeStruct((BATCH, NUM_OUT), jnp.float32),
    )(h2w, W_out, b_out.reshape(1, NUM_OUT))


_GDN = lax.GatherDimensionNumbers(
    offset_dims=(), collapsed_slice_dims=(0,), start_index_map=(0,))


def _lane_bcast(vec, k):
    si = jnp.full((L, 1), k, jnp.int32)
    return lax.gather(vec, si, _GDN, (1,),
                      mode=lax.GatherScatterMode.PROMISE_IN_BOUNDS)


def _cond_sc(hT, idx_f, w_f, b_vec, interpret=False):
    mesh = plsc.VectorSubcoreMesh(core_axis_name="c", subcore_axis_name="s",
                                  num_cores=NC, num_subcores=NS)

    NBUF = 4
    G = 2
    NP = RPW // G

    @functools.partial(
        pl.kernel,
        out_type=jax.ShapeDtypeStruct((NUM_MID, BATCH // 2), jnp.int32),
        mesh=mesh,
        interpret=interpret,
        scratch_types=[
            pltpu.VMEM((RPW * FAN_IN,), jnp.int32),
            pltpu.VMEM((RPW * FAN_IN,), jnp.float32),
            pltpu.VMEM((RPW,), jnp.float32),
            pltpu.VMEM((NBUF, G * FAN_IN, BATCH // 2), jnp.int32),
            pltpu.VMEM((NBUF, G, BATCH // 2), jnp.int32),
            [pltpu.SemaphoreType.DMA] * NBUF,
            [pltpu.SemaphoreType.DMA] * NBUF,
        ],
    )
    def k(hT_hbm, idx_hbm, w_hbm, b_hbm, out_hbm,
          idx_v, w_v, b_v, rows_v, ostage_v, gsems, osems):
        wid = lax.axis_index("s") * NC + lax.axis_index("c")
        base = wid * RPW
        pltpu.sync_copy(idx_hbm.at[pl.ds(base * FAN_IN, RPW * FAN_IN)], idx_v)
        pltpu.sync_copy(w_hbm.at[pl.ds(base * FAN_IN, RPW * FAN_IN)], w_v)
        pltpu.sync_copy(b_hbm.at[pl.ds(base, RPW)], b_v)

        def gather_idx(p):
            return idx_v.at[pl.ds(p * G * FAN_IN, G * FAN_IN)]

        for b in range(NBUF):
            pltpu.async_copy(hT_hbm.at[gather_idx(b)], rows_v.at[b], gsems[b])

        def grp_body(p0, carry):
            for b in range(NBUF):
                p = p0 + b
                pltpu.make_async_copy(hT_hbm.at[gather_idx(p)], rows_v.at[b],
                                      gsems[b]).wait()
                @pl.when(p0 >= NBUF)
                def _():
                    pltpu.make_async_copy(ostage_v.at[b],
                                          out_hbm.at[pl.ds(base, G)],
                                          osems[b]).wait()

                for g in range(G):
                    j = p * G + g
                    bblk = b_v[pl.ds(jnp.bitwise_and(j, -L), L)]
                    bvec = _lane_bcast(bblk, jnp.bitwise_and(j, L - 1))
                    wvec = w_v[pl.ds(j * FAN_IN, FAN_IN)]
                    wks = [_lane_bcast(wvec, k) for k in range(FAN_IN)]

                    @plsc.parallel_loop(0, BATCH // (2 * L), unroll=2)
                    def _(c):
                        acc_e = bvec
                        acc_o = bvec
                        for k in range(FAN_IN):
                            w = rows_v[b, g * FAN_IN + k, pl.ds(c * L, L)]
                            e = lax.bitcast_convert_type(
                                lax.shift_left(w, 16), jnp.float32)
                            o = lax.bitcast_convert_type(
                                w & jnp.int32(-65536), jnp.float32)
                            acc_e = acc_e + wks[k] * e
                            acc_o = acc_o + wks[k] * o
                        be = lax.bitcast_convert_type(
                            jnp.maximum(acc_e, 0.0), jnp.int32)
                        bo = lax.bitcast_convert_type(
                            jnp.maximum(acc_o, 0.0), jnp.int32)
                        half = jnp.int32(0x8000)
                        word = lax.shift_right_logical(be + half, 16) | (
                            (bo + half) & jnp.int32(-65536))
                        ostage_v[b, g, pl.ds(c * L, L)] = word

                @pl.when(p0 < NP - NBUF)
                def _():
                    pltpu.async_copy(hT_hbm.at[gather_idx(p + NBUF)],
                                     rows_v.at[b], gsems[b])

                pltpu.async_copy(ostage_v.at[b],
                                 out_hbm.at[pl.ds(base + p * G, G)],
                                 osems[b])
            return carry

        lax.fori_loop(0, NP // NBUF, lambda i, c: grp_body(i * NBUF, c), 0)
        for b in range(NBUF):
            pltpu.make_async_copy(ostage_v.at[b], out_hbm.at[pl.ds(base, G)],
                                  osems[b]).wait()

    return k(hT, idx_f, w_f, b_vec)


def kernel(x, W_in, b_in, W_mid0, b_mid0, W_mid1, b_mid1, W_out, b_out,
           indx_seqs):
    idx_f = indx_seqs.reshape(-1)

    h0w = _mm1(W_in, x, b_in)
    h1w = _cond_sc(h0w, idx_f, W_mid0.reshape(-1), b_mid0)
    h2w = _cond_sc(h1w, idx_f, W_mid1.reshape(-1), b_mid1)
    return _mm2(h2w, W_out, b_out)


# speedup vs baseline: 1.1144x; 1.1144x over previous
import functools

import jax
import jax.numpy as jnp
from jax import lax
from jax.experimental import pallas as pl
from jax.experimental.pallas import tpu as pltpu
from jax.experimental.pallas import tpu_sc as plsc

NUM_IN = 1024
NUM_OUT = 1024
NUM_MID = 4096
FAN_IN = 16
BATCH = 1024

NC = 2
NS = 16
NW = NC * NS
RPW = NUM_MID // NW
L = 16
NCHUNK = BATCH // L


_HALF = BATCH // 2


def _pack_words(a, b):
    ba = lax.bitcast_convert_type(a, jnp.int32)
    bb = lax.bitcast_convert_type(b, jnp.int32)
    half = jnp.int32(0x8000)
    return lax.shift_right_logical(ba + half, 16) | (
        (bb + half) & jnp.int32(-65536))


def _mm1_body(w_ref, x_ref, b_ref, o_ref):
    acc = lax.dot_general(w_ref[...].astype(jnp.bfloat16),
                          x_ref[...].astype(jnp.bfloat16),
                          (((1,), (1,)), ((), ())),
                          preferred_element_type=jnp.float32)
    r = jnp.maximum(acc + b_ref[...], 0.0)
    o_ref[...] = _pack_words(r[:, :_HALF], r[:, _HALF:])


def _mm1(W_in, x, b_in):
    BM = 1024
    return pl.pallas_call(
        _mm1_body,
        grid=(NUM_MID // BM,),
        in_specs=[
            pl.BlockSpec((BM, NUM_IN), lambda i: (i, 0)),
            pl.BlockSpec((BATCH, NUM_IN), lambda i: (0, 0)),
            pl.BlockSpec((BM, 1), lambda i: (i, 0)),
        ],
        out_specs=pl.BlockSpec((BM, _HALF), lambda i: (i, 0)),
        out_shape=jax.ShapeDtypeStruct((NUM_MID, _HALF), jnp.int32),
    )(W_in, x, b_in.reshape(NUM_MID, 1))


def _mm2_body(h_ref, w_ref, b_ref, o_ref):
    w = h_ref[...]
    e = lax.bitcast_convert_type(lax.shift_left(w, 16), jnp.float32)
    o = lax.bitcast_convert_type(w & jnp.int32(-65536), jnp.float32)
    wb = w_ref[...].astype(jnp.bfloat16)
    acc_e = lax.dot_general(e.astype(jnp.bfloat16), wb,
                            (((0,), (1,)), ((), ())),
                            preferred_element_type=jnp.float32)
    acc_o = lax.dot_general(o.astype(jnp.bfloat16), wb,
                            (((0,), (1,)), ((), ())),
                            preferred_element_type=jnp.float32)
    o_ref[:_HALF, :] = acc_e + b_ref[...]
    o_ref[_HALF:, :] = acc_o + b_ref[...]


def _mm2(h2w, W_out, b_out):
    BO = 256
    return pl.pallas_call(
        _mm2_body,
        grid=(NUM_OUT // BO,),
        in_specs=[
            pl.BlockSpec((NUM_MID, _HALF), lambda i: (0, 0)),
            pl.BlockSpec((BO, NUM_MID), lambda i: (i, 0)),
            pl.BlockSpec((1, BO), lambda i: (0, i)),
        ],
        out_specs=pl.BlockSpec((BATCH, BO), lambda i: (0, i)),
        out_shape=jax.ShapeDtypeStruct((BATCH, NUM_OUT), jnp.float32),
    )(h2w, W_out, b_out.reshape(1, NUM_OUT))


_GDN = lax.GatherDimensionNumbers(
    offset_dims=(), collapsed_slice_dims=(0,), start_index_map=(0,))


def _lane_bcast(vec, k):
    si = jnp.full((L, 1), k, jnp.int32)
    return lax.gather(vec, si, _GDN, (1,),
                      mode=lax.GatherScatterMode.PROMISE_IN_BOUNDS)


def _cond_sc(hT, idx_f, w_f, b_vec, interpret=False):
    mesh = plsc.VectorSubcoreMesh(core_axis_name="c", subcore_axis_name="s",
                                  num_cores=NC, num_subcores=NS)

    NBUF = 4
    G = 1
    NP = RPW // G

    @functools.partial(
        pl.kernel,
        out_type=jax.ShapeDtypeStruct((NUM_MID, BATCH // 2), jnp.int32),
        mesh=mesh,
        interpret=interpret,
        scratch_types=[
            pltpu.VMEM((RPW * FAN_IN,), jnp.int32),
            pltpu.VMEM((RPW * FAN_IN,), jnp.float32),
            pltpu.VMEM((RPW,), jnp.float32),
            pltpu.VMEM((NBUF, G * FAN_IN, BATCH // 2), jnp.int32),
            pltpu.VMEM((NBUF, G, BATCH // 2), jnp.int32),
            [pltpu.SemaphoreType.DMA] * NBUF,
            [pltpu.SemaphoreType.DMA] * NBUF,
        ],
    )
    def k(hT_hbm, idx_hbm, w_hbm, b_hbm, out_hbm,
          idx_v, w_v, b_v, rows_v, ostage_v, gsems, osems):
        wid = lax.axis_index("s") * NC + lax.axis_index("c")
        base = wid * RPW
        pltpu.sync_copy(idx_hbm.at[pl.ds(base * FAN_IN, RPW * FAN_IN)], idx_v)
        pltpu.sync_copy(w_hbm.at[pl.ds(base * FAN_IN, RPW * FAN_IN)], w_v)
        pltpu.sync_copy(b_hbm.at[pl.ds(base, RPW)], b_v)

        def gather_idx(p):
            return idx_v[pl.ds(p * G * FAN_IN, G * FAN_IN)]

        for b in range(NBUF):
            pltpu.async_copy(hT_hbm.at[gather_idx(b)], rows_v.at[b], gsems[b])

        def grp_body(p0, carry):
            for b in range(NBUF):
                p = p0 + b
                pltpu.make_async_copy(hT_hbm.at[gather_idx(p)], rows_v.at[b],
                                      gsems[b]).wait()
                @pl.when(p0 >= NBUF)
                def _():
                    pltpu.make_async_copy(ostage_v.at[b],
                                          out_hbm.at[pl.ds(base, G)],
                                          osems[b]).wait()

                for g in range(G):
                    j = p * G + g
                    bblk = b_v[pl.ds(jnp.bitwise_and(j, -L), L)]
                    bvec = _lane_bcast(bblk, jnp.bitwise_and(j, L - 1))
                    wvec = w_v[pl.ds(j * FAN_IN, FAN_IN)]
                    wks = [_lane_bcast(wvec, k) for k in range(FAN_IN)]

                    @plsc.parallel_loop(0, BATCH // (2 * L), unroll=2)
                    def _(c):
                        acc_e = bvec
                        acc_o = bvec
                        for k in range(FAN_IN):
                            w = rows_v[b, g * FAN_IN + k, pl.ds(c * L, L)]
                            e = lax.bitcast_convert_type(
                                lax.shift_left(w, 16), jnp.float32)
                            o = lax.bitcast_convert_type(
                                w & jnp.int32(-65536), jnp.float32)
                            acc_e = acc_e + wks[k] * e
                            acc_o = acc_o + wks[k] * o
                        be = lax.bitcast_convert_type(
                            jnp.maximum(acc_e, 0.0), jnp.int32)
                        bo = lax.bitcast_convert_type(
                            jnp.maximum(acc_o, 0.0), jnp.int32)
                        half = jnp.int32(0x8000)
                        word = lax.shift_right_logical(be + half, 16) | (
                            (bo + half) & jnp.int32(-65536))
                        ostage_v[b, g, pl.ds(c * L, L)] = word

                @pl.when(p0 < NP - NBUF)
                def _():
                    pltpu.async_copy(hT_hbm.at[gather_idx(p + NBUF)],
                                     rows_v.at[b], gsems[b])

                pltpu.async_copy(ostage_v.at[b],
                                 out_hbm.at[pl.ds(base + p * G, G)],
                                 osems[b])
            return carry

        lax.fori_loop(0, NP // NBUF, lambda i, c: grp_body(i * NBUF, c), 0)
        for b in range(NBUF):
            pltpu.make_async_copy(ostage_v.at[b], out_hbm.at[pl.ds(base, G)],
                                  osems[b]).wait()

    return k(hT, idx_f, w_f, b_vec)


def kernel(x, W_in, b_in, W_mid0, b_mid0, W_mid1, b_mid1, W_out, b_out,
           indx_seqs):
    idx_f = indx_seqs.reshape(-1)

    h0w = _mm1(W_in, x, b_in)
    h1w = _cond_sc(h0w, idx_f, W_mid0.reshape(-1), b_mid0)
    h2w = _cond_sc(h1w, idx_f, W_mid1.reshape(-1), b_mid1)
    return _mm2(h2w, W_out, b_out)
